# baseline (speedup 1.0000x reference)
import functools

import jax
import jax.numpy as jnp
from jax import lax
from jax.experimental import pallas as pl
from jax.experimental.pallas import tpu as pltpu
from jax.experimental.pallas import tpu_sc as plsc

N_NODES = 10000
N_EDGES = 320000
N_GRAPHS = 128

NC = 2
NS = 16
NW = NC * NS
K = 128
NCH = 80
NCH_PAD = NW * NCH
E_PAD = NCH_PAD * K
RPT = 632
N_PAD = RPT * NS
HRPT = 625
G = 4


def _make_sc_agg(D):
  mesh = plsc.VectorSubcoreMesh(core_axis_name="c", subcore_axis_name="s",
                                num_cores=NC, num_subcores=NS)

  @functools.partial(
      pl.kernel,
      out_type=jax.ShapeDtypeStruct((NC, N_PAD, D), jnp.float32),
      mesh=mesh,
      compiler_params=pltpu.CompilerParams(use_tc_tiling_on_sc=False),
      scratch_types=[
          pltpu.VMEM((NCH, K), jnp.int32),
          pltpu.VMEM((NCH, K), jnp.int32),
          [pltpu.VMEM((K, D), jnp.float32) for _ in range(G)],
          [pltpu.VMEM((K, D), jnp.float32) for _ in range(G)],
          pltpu.VMEM_SHARED((N_NODES, D), jnp.float32),
          pltpu.VMEM_SHARED((N_PAD, D), jnp.float32),
          pltpu.SemaphoreType.DMA,
          pltpu.SemaphoreType.DMA,
          pltpu.SemaphoreType.DMA,
          pltpu.SemaphoreType.DMA,
      ],
  )
  def agg(h_hbm, src_hbm, dst_hbm, zeros_hbm, out_hbm,
          srcv, dstv, bufsa, bufsb, htab, accum, gsema, gsemb, ssema, ssemb):
    c = lax.axis_index("c")
    s = lax.axis_index("s")
    wid = c * NS + s

    row0 = s * RPT
    pltpu.sync_copy(zeros_hbm, accum.at[pl.ds(row0, RPT)])
    pltpu.sync_copy(h_hbm.at[pl.ds(s * HRPT, HRPT)],
                    htab.at[pl.ds(s * HRPT, HRPT)])

    pltpu.sync_copy(src_hbm.at[pl.ds(wid * NCH, NCH)], srcv)
    pltpu.sync_copy(dst_hbm.at[pl.ds(wid * NCH, NCH)], dstv)
    plsc.subcore_barrier()

    pltpu.async_copy(htab.at[srcv.at[0]], bufsa[0], gsema)

    def body(i, _):
      j = 2 * i
      gather0 = pltpu.make_async_copy(htab.at[srcv.at[j]], bufsa[0], gsema)
      gather1 = pltpu.async_copy(htab.at[srcv.at[j + 1]], bufsb[0], gsemb)
      gather0.wait()
      pltpu.async_copy(bufsa[0], accum.at[dstv.at[j]], ssema, add=True)
      gather1.wait()
      pltpu.async_copy(bufsb[0], accum.at[dstv.at[j + 1]], ssemb, add=True)
      pltpu.make_async_copy(bufsa[0], accum.at[dstv.at[j]], ssema).wait()

      @pl.when(j + 2 < NCH)
      def _():
        pltpu.async_copy(htab.at[srcv.at[j + 2]], bufsa[0], gsema)

      pltpu.make_async_copy(bufsb[0], accum.at[dstv.at[j + 1]], ssemb).wait()
      return ()

    lax.fori_loop(0, NCH // 2, body, (), unroll=False)

    plsc.subcore_barrier()
    pltpu.sync_copy(accum.at[pl.ds(row0, RPT)],
                    out_hbm.at[c, pl.ds(row0, RPT)])

  return agg


def _mlp_bn(m, W1_ref, b1_ref, g1_ref, be1_ref, W2_ref, b2_ref, go_ref,
            bo_ref):
  t = jnp.dot(m, W1_ref[...], preferred_element_type=jnp.float32)
  t = t + b1_ref[...]
  mu = jnp.mean(t, axis=0, keepdims=True)
  var = jnp.mean(jnp.square(t - mu), axis=0, keepdims=True)
  t = (t - mu) * lax.rsqrt(var + 1e-5) * g1_ref[...] + be1_ref[...]
  t = jnp.maximum(t, 0.0)
  u = jnp.dot(t, W2_ref[...], preferred_element_type=jnp.float32)
  u = u + b2_ref[...]
  mu2 = jnp.mean(u, axis=0, keepdims=True)
  var2 = jnp.mean(jnp.square(u - mu2), axis=0, keepdims=True)
  u = (u - mu2) * lax.rsqrt(var2 + 1e-5) * go_ref[...] + bo_ref[...]
  return jnp.maximum(u, 0.0)


def _tc_layer(h, p, W1, b1, g1, be1, W2, b2, go, bo):
  n = h.shape[0]
  dout = W2.shape[1]

  def body(h_ref, p_ref, W1_ref, b1_ref, g1_ref, be1_ref, W2_ref, b2_ref,
           go_ref, bo_ref, o_ref):
    m = h_ref[...] + p_ref[0, :N_NODES, :] + p_ref[1, :N_NODES, :]
    o_ref[...] = _mlp_bn(m, W1_ref, b1_ref, g1_ref, be1_ref, W2_ref, b2_ref,
                         go_ref, bo_ref)

  return pl.pallas_call(
      body,
      out_shape=jax.ShapeDtypeStruct((n, dout), jnp.float32),
  )(h, p, W1, b1.reshape(1, -1), g1.reshape(1, -1), be1.reshape(1, -1),
    W2, b2.reshape(1, -1), go.reshape(1, -1), bo.reshape(1, -1))


def _tc_layer0(h, plo, phi, W1, b1, g1, be1, W2, b2, go, bo):
  n = h.shape[0]
  dout = W2.shape[1]

  def body(h_ref, plo_ref, phi_ref, W1_ref, b1_ref, g1_ref, be1_ref, W2_ref,
           b2_ref, go_ref, bo_ref, o_ref):
    agg = jnp.concatenate(
        [plo_ref[0, :N_NODES, :] + plo_ref[1, :N_NODES, :],
         phi_ref[0, :N_NODES, :] + phi_ref[1, :N_NODES, :]], axis=1)
    m = h_ref[...] + agg
    o_ref[...] = _mlp_bn(m, W1_ref, b1_ref, g1_ref, be1_ref, W2_ref, b2_ref,
                         go_ref, bo_ref)

  return pl.pallas_call(
      body,
      out_shape=jax.ShapeDtypeStruct((n, dout), jnp.float32),
  )(h, plo, phi, W1, b1.reshape(1, -1), g1.reshape(1, -1),
    be1.reshape(1, -1), W2, b2.reshape(1, -1), go.reshape(1, -1),
    bo.reshape(1, -1))


def _tc_layer_readout(h, p, batch_row, W1, b1, g1, be1, W2, b2, go, bo):
  n = h.shape[0]
  dout = W2.shape[1]

  def body(h_ref, p_ref, batch_ref, W1_ref, b1_ref, g1_ref, be1_ref, W2_ref,
           b2_ref, go_ref, bo_ref, o_ref):
    m = h_ref[...] + p_ref[0, :N_NODES, :] + p_ref[1, :N_NODES, :]
    hout = _mlp_bn(m, W1_ref, b1_ref, g1_ref, be1_ref, W2_ref, b2_ref,
                   go_ref, bo_ref)
    gids = lax.broadcasted_iota(jnp.int32, (N_GRAPHS, n), 0)
    onehot = jnp.where(gids == batch_ref[...], 1.0, 0.0).astype(jnp.float32)
    sums = jnp.dot(onehot, hout, preferred_element_type=jnp.float32)
    counts = jnp.sum(onehot, axis=1, keepdims=True)
    o_ref[...] = sums / jnp.maximum(counts, 1.0)

  return pl.pallas_call(
      body,
      out_shape=jax.ShapeDtypeStruct((N_GRAPHS, dout), jnp.float32),
  )(h, p, batch_row, W1, b1.reshape(1, -1), g1.reshape(1, -1),
    be1.reshape(1, -1), W2, b2.reshape(1, -1), go.reshape(1, -1),
    bo.reshape(1, -1))


_agg64 = _make_sc_agg(64)


def kernel(x, edge_index, batch,
           W1_0, b1_0, g1_0, be1_0, W2_0, b2_0, go_0, bo_0,
           W1_1, b1_1, g1_1, be1_1, W2_1, b2_1, go_1, bo_1,
           W1_2, b1_2, g1_2, be1_2, W2_2, b2_2, go_2, bo_2):
  npad = E_PAD - N_EDGES
  src = jnp.concatenate(
      [edge_index[0].astype(jnp.int32),
       jnp.zeros((npad,), jnp.int32)]).reshape(NCH_PAD, K)
  dst = jnp.concatenate(
      [edge_index[1].astype(jnp.int32),
       jnp.full((npad,), N_NODES, jnp.int32)]).reshape(NCH_PAD, K)
  batch_row = batch.astype(jnp.int32).reshape(1, N_NODES)
  z64 = jnp.zeros((RPT, 64), jnp.float32)

  x_lo = x[:, :64]
  x_hi = x[:, 64:]
  p0_lo = _agg64(x_lo, src, dst, z64)
  p0_hi = _agg64(x_hi, src, dst, z64)
  h1 = _tc_layer0(x, p0_lo, p0_hi, W1_0, b1_0, g1_0, be1_0, W2_0, b2_0,
                  go_0, bo_0)
  p1 = _agg64(h1, src, dst, z64)
  h2 = _tc_layer(h1, p1, W1_1, b1_1, g1_1, be1_1, W2_1, b2_1, go_1, bo_1)
  p2 = _agg64(h2, src, dst, z64)
  return _tc_layer_readout(h2, p2, batch_row, W1_2, b1_2, g1_2, be1_2,
                           W2_2, b2_2, go_2, bo_2)


# speedup vs baseline: 1.1319x; 1.1319x over previous
import functools

import jax
import jax.numpy as jnp
from jax import lax
from jax.experimental import pallas as pl
from jax.experimental.pallas import tpu as pltpu
from jax.experimental.pallas import tpu_sc as plsc

N_NODES = 10000
N_EDGES = 320000
N_GRAPHS = 128

NC = 2
NS = 16
NW = NC * NS
K = 128
NCHTOT = N_EDGES // K
NCH = NCHTOT // NW
NCH_XTRA = NCHTOT - NCH * NW
RPT = 632
N_PAD = RPT * NS
HRPT = 625


def _make_sc_agg(col0):
  D = 64
  mesh = plsc.VectorSubcoreMesh(core_axis_name="c", subcore_axis_name="s",
                                num_cores=NC, num_subcores=NS)

  @functools.partial(
      pl.kernel,
      out_type=jax.ShapeDtypeStruct((NC, N_PAD, D), jnp.float32),
      mesh=mesh,
      compiler_params=pltpu.CompilerParams(use_tc_tiling_on_sc=False),
      scratch_types=[
          pltpu.VMEM((NCH + 1, K), jnp.int32),
          pltpu.VMEM((NCH + 1, K), jnp.int32),
          pltpu.VMEM((K, D), jnp.float32),
          pltpu.VMEM((K, D), jnp.float32),
          pltpu.VMEM_SHARED((N_NODES, D), jnp.float32),
          pltpu.VMEM_SHARED((N_PAD, D), jnp.float32),
          pltpu.SemaphoreType.DMA,
          pltpu.SemaphoreType.DMA,
      ],
  )
  def agg(h_hbm, ei_hbm, zeros_hbm, out_hbm,
          srcv, dstv, bufa, bufb, htab, accum, sema, semb):
    c = lax.axis_index("c")
    s = lax.axis_index("s")
    wid = c * NS + s

    row0 = s * RPT
    pltpu.sync_copy(zeros_hbm, accum.at[pl.ds(row0, RPT)])
    pltpu.sync_copy(
        h_hbm.at[pl.ds(s * HRPT, HRPT), pl.ds(col0, D)],
        htab.at[pl.ds(s * HRPT, HRPT)])

    base = wid * NCH + jnp.minimum(wid, NCH_XTRA)
    extra = wid < NCH_XTRA
    pltpu.sync_copy(ei_hbm.at[0, pl.ds(base, NCH)], srcv.at[pl.ds(0, NCH)])
    pltpu.sync_copy(ei_hbm.at[1, pl.ds(base, NCH)], dstv.at[pl.ds(0, NCH)])

    @pl.when(extra)
    def _():
      pltpu.sync_copy(ei_hbm.at[0, pl.ds(base + NCH, 1)],
                      srcv.at[pl.ds(NCH, 1)])
      pltpu.sync_copy(ei_hbm.at[1, pl.ds(base + NCH, 1)],
                      dstv.at[pl.ds(NCH, 1)])

    plsc.subcore_barrier()

    pltpu.async_copy(htab.at[srcv.at[0]], bufa, sema)

    def body(i, _):
      j = 2 * i
      gather0 = pltpu.make_async_copy(htab.at[srcv.at[j]], bufa, sema)
      gather1 = pltpu.async_copy(htab.at[srcv.at[j + 1]], bufb, semb)
      gather0.wait()
      pltpu.sync_copy(bufa, accum.at[dstv.at[j]], add=True)

      @pl.when(j + 2 < NCH)
      def _():
        pltpu.async_copy(htab.at[srcv.at[j + 2]], bufa, sema)

      gather1.wait()
      pltpu.sync_copy(bufb, accum.at[dstv.at[j + 1]], add=True)
      return ()

    lax.fori_loop(0, NCH // 2, body, (), unroll=False)

    @pl.when(extra)
    def _():
      pltpu.async_copy(htab.at[srcv.at[NCH]], bufa, sema).wait()
      pltpu.sync_copy(bufa, accum.at[dstv.at[NCH]], add=True)

    plsc.subcore_barrier()
    pltpu.sync_copy(accum.at[pl.ds(row0, RPT)],
                    out_hbm.at[c, pl.ds(row0, RPT)])

  return agg


def _mlp_bn(m, W1_ref, b1_ref, g1_ref, be1_ref, W2_ref, b2_ref, go_ref,
            bo_ref):
  t = jnp.dot(m, W1_ref[...], preferred_element_type=jnp.float32)
  t = t + b1_ref[...]
  mu = jnp.mean(t, axis=0, keepdims=True)
  var = jnp.mean(jnp.square(t - mu), axis=0, keepdims=True)
  t = (t - mu) * lax.rsqrt(var + 1e-5) * g1_ref[...] + be1_ref[...]
  t = jnp.maximum(t, 0.0)
  u = jnp.dot(t, W2_ref[...], preferred_element_type=jnp.float32)
  u = u + b2_ref[...]
  mu2 = jnp.mean(u, axis=0, keepdims=True)
  var2 = jnp.mean(jnp.square(u - mu2), axis=0, keepdims=True)
  u = (u - mu2) * lax.rsqrt(var2 + 1e-5) * go_ref[...] + bo_ref[...]
  return jnp.maximum(u, 0.0)


def _tc_layer(h, p, W1, b1, g1, be1, W2, b2, go, bo):
  n = h.shape[0]
  dout = W2.shape[1]

  def body(h_ref, p_ref, W1_ref, b1_ref, g1_ref, be1_ref, W2_ref, b2_ref,
           go_ref, bo_ref, o_ref):
    m = h_ref[...] + p_ref[0, :N_NODES, :] + p_ref[1, :N_NODES, :]
    o_ref[...] = _mlp_bn(m, W1_ref, b1_ref, g1_ref, be1_ref, W2_ref, b2_ref,
                         go_ref, bo_ref)

  return pl.pallas_call(
      body,
      out_shape=jax.ShapeDtypeStruct((n, dout), jnp.float32),
  )(h, p, W1, b1.reshape(1, -1), g1.reshape(1, -1), be1.reshape(1, -1),
    W2, b2.reshape(1, -1), go.reshape(1, -1), bo.reshape(1, -1))


def _tc_layer0(h, plo, phi, W1, b1, g1, be1, W2, b2, go, bo):
  n = h.shape[0]
  dout = W2.shape[1]

  def body(h_ref, plo_ref, phi_ref, W1_ref, b1_ref, g1_ref, be1_ref, W2_ref,
           b2_ref, go_ref, bo_ref, o_ref):
    agg = jnp.concatenate(
        [plo_ref[0, :N_NODES, :] + plo_ref[1, :N_NODES, :],
         phi_ref[0, :N_NODES, :] + phi_ref[1, :N_NODES, :]], axis=1)
    m = h_ref[...] + agg
    o_ref[...] = _mlp_bn(m, W1_ref, b1_ref, g1_ref, be1_ref, W2_ref, b2_ref,
                         go_ref, bo_ref)

  return pl.pallas_call(
      body,
      out_shape=jax.ShapeDtypeStruct((n, dout), jnp.float32),
  )(h, plo, phi, W1, b1.reshape(1, -1), g1.reshape(1, -1),
    be1.reshape(1, -1), W2, b2.reshape(1, -1), go.reshape(1, -1),
    bo.reshape(1, -1))


def _tc_layer_readout(h, p, batch_row, W1, b1, g1, be1, W2, b2, go, bo):
  n = h.shape[0]
  dout = W2.shape[1]

  def body(h_ref, p_ref, batch_ref, W1_ref, b1_ref, g1_ref, be1_ref, W2_ref,
           b2_ref, go_ref, bo_ref, o_ref):
    m = h_ref[...] + p_ref[0, :N_NODES, :] + p_ref[1, :N_NODES, :]
    hout = _mlp_bn(m, W1_ref, b1_ref, g1_ref, be1_ref, W2_ref, b2_ref,
                   go_ref, bo_ref)
    gids = lax.broadcasted_iota(jnp.int32, (N_GRAPHS, n), 0)
    onehot = jnp.where(gids == batch_ref[...], 1.0, 0.0).astype(jnp.float32)
    sums = jnp.dot(onehot, hout, preferred_element_type=jnp.float32)
    counts = jnp.sum(onehot, axis=1, keepdims=True)
    o_ref[...] = sums / jnp.maximum(counts, 1.0)

  return pl.pallas_call(
      body,
      out_shape=jax.ShapeDtypeStruct((N_GRAPHS, dout), jnp.float32),
  )(h, p, batch_row, W1, b1.reshape(1, -1), g1.reshape(1, -1),
    be1.reshape(1, -1), W2, b2.reshape(1, -1), go.reshape(1, -1),
    bo.reshape(1, -1))


_agg_lo = _make_sc_agg(0)
_agg_hi = _make_sc_agg(64)


def kernel(x, edge_index, batch,
           W1_0, b1_0, g1_0, be1_0, W2_0, b2_0, go_0, bo_0,
           W1_1, b1_1, g1_1, be1_1, W2_1, b2_1, go_1, bo_1,
           W1_2, b1_2, g1_2, be1_2, W2_2, b2_2, go_2, bo_2):
  ei = edge_index.astype(jnp.int32).reshape(2, NCHTOT, K)
  batch_row = batch.astype(jnp.int32).reshape(1, N_NODES)
  z64 = jnp.zeros((RPT, 64), jnp.float32)

  p0_lo = _agg_lo(x, ei, z64)
  p0_hi = _agg_hi(x, ei, z64)
  h1 = _tc_layer0(x, p0_lo, p0_hi, W1_0, b1_0, g1_0, be1_0, W2_0, b2_0,
                  go_0, bo_0)
  p1 = _agg_lo(h1, ei, z64)
  h2 = _tc_layer(h1, p1, W1_1, b1_1, g1_1, be1_1, W2_1, b2_1, go_1, bo_1)
  p2 = _agg_lo(h2, ei, z64)
  return _tc_layer_readout(h2, p2, batch_row, W1_2, b1_2, g1_2, be1_2,
                           W2_2, b2_2, go_2, bo_2)


# speedup vs baseline: 1.1916x; 1.0528x over previous
import functools

import jax
import jax.numpy as jnp
from jax import lax
from jax.experimental import pallas as pl
from jax.experimental.pallas import tpu as pltpu
from jax.experimental.pallas import tpu_sc as plsc

N_NODES = 10000
N_EDGES = 320000
N_GRAPHS = 128

NC = 2
NS = 16
NW = NC * NS
K = 128
NCHTOT = N_EDGES // K
NCH = NCHTOT // NW
NCH_XTRA = NCHTOT - NCH * NW
RPT = 632
N_PAD = RPT * NS
HRPT = 625


def _make_sc_agg(col0):
  D = 64
  mesh = plsc.VectorSubcoreMesh(core_axis_name="c", subcore_axis_name="s",
                                num_cores=NC, num_subcores=NS)

  @functools.partial(
      pl.kernel,
      out_type=jax.ShapeDtypeStruct((NC, N_PAD, D), jnp.float32),
      mesh=mesh,
      compiler_params=pltpu.CompilerParams(use_tc_tiling_on_sc=False),
      scratch_types=[
          pltpu.VMEM((NCH + 1, K), jnp.int32),
          pltpu.VMEM((NCH + 1, K), jnp.int32),
          pltpu.VMEM((K, D), jnp.float32),
          pltpu.VMEM((K, D), jnp.float32),
          pltpu.VMEM_SHARED((N_NODES, D), jnp.float32),
          pltpu.VMEM_SHARED((N_PAD, D), jnp.float32),
          pltpu.SemaphoreType.DMA,
          pltpu.SemaphoreType.DMA,
      ],
  )
  def agg(h_hbm, ei_hbm, zeros_hbm, out_hbm,
          srcv, dstv, bufa, bufb, htab, accum, sema, semb):
    c = lax.axis_index("c")
    s = lax.axis_index("s")
    wid = c * NS + s

    row0 = s * RPT
    pltpu.sync_copy(zeros_hbm, accum.at[pl.ds(row0, RPT)])
    pltpu.sync_copy(
        h_hbm.at[pl.ds(s * HRPT, HRPT), pl.ds(col0, D)],
        htab.at[pl.ds(s * HRPT, HRPT)])

    base = wid * NCH + jnp.minimum(wid, NCH_XTRA)
    extra = wid < NCH_XTRA
    pltpu.sync_copy(ei_hbm.at[0, pl.ds(base, NCH)], srcv.at[pl.ds(0, NCH)])
    pltpu.sync_copy(ei_hbm.at[1, pl.ds(base, NCH)], dstv.at[pl.ds(0, NCH)])

    @pl.when(extra)
    def _():
      pltpu.sync_copy(ei_hbm.at[0, pl.ds(base + NCH, 1)],
                      srcv.at[pl.ds(NCH, 1)])
      pltpu.sync_copy(ei_hbm.at[1, pl.ds(base + NCH, 1)],
                      dstv.at[pl.ds(NCH, 1)])

    plsc.subcore_barrier()

    pltpu.async_copy(htab.at[srcv.at[0]], bufa, sema)

    def body(i, _):
      j = 2 * i
      gather0 = pltpu.make_async_copy(htab.at[srcv.at[j]], bufa, sema)
      gather1 = pltpu.async_copy(htab.at[srcv.at[j + 1]], bufb, semb)
      gather0.wait()
      pltpu.sync_copy(bufa, accum.at[dstv.at[j]], add=True)

      @pl.when(j + 2 < NCH)
      def _():
        pltpu.async_copy(htab.at[srcv.at[j + 2]], bufa, sema)

      gather1.wait()
      pltpu.sync_copy(bufb, accum.at[dstv.at[j + 1]], add=True)
      return ()

    lax.fori_loop(0, NCH // 2, body, (), unroll=False)

    @pl.when(extra)
    def _():
      pltpu.async_copy(htab.at[srcv.at[NCH]], bufa, sema).wait()
      pltpu.sync_copy(bufa, accum.at[dstv.at[NCH]], add=True)

    plsc.subcore_barrier()
    pltpu.sync_copy(accum.at[pl.ds(row0, RPT)],
                    out_hbm.at[c, pl.ds(row0, RPT)])

  return agg


def _mlp_bn(m, W1_ref, b1_ref, g1_ref, be1_ref, W2_ref, b2_ref, go_ref,
            bo_ref):
  t = jnp.dot(m, W1_ref[...], preferred_element_type=jnp.float32)
  t = t + b1_ref[...]
  mu = jnp.mean(t, axis=0, keepdims=True)
  var = jnp.mean(jnp.square(t - mu), axis=0, keepdims=True)
  t = (t - mu) * lax.rsqrt(var + 1e-5) * g1_ref[...] + be1_ref[...]
  t = jnp.maximum(t, 0.0)
  u = jnp.dot(t, W2_ref[...], preferred_element_type=jnp.float32)
  u = u + b2_ref[...]
  mu2 = jnp.mean(u, axis=0, keepdims=True)
  var2 = jnp.mean(jnp.square(u - mu2), axis=0, keepdims=True)
  u = (u - mu2) * lax.rsqrt(var2 + 1e-5) * go_ref[...] + bo_ref[...]
  return jnp.maximum(u, 0.0)


def _unpack_agg(p_ref):
  q = p_ref[0] + p_ref[1]
  return jnp.concatenate([q[:, :64], q[:, 64:]], axis=0)[:N_NODES, :]


def _tc_layer(h, p, W1, b1, g1, be1, W2, b2, go, bo):
  n = h.shape[0]
  dout = W2.shape[1]

  def body(h_ref, p_ref, W1_ref, b1_ref, g1_ref, be1_ref, W2_ref, b2_ref,
           go_ref, bo_ref, o_ref):
    m = h_ref[...] + _unpack_agg(p_ref)
    o_ref[...] = _mlp_bn(m, W1_ref, b1_ref, g1_ref, be1_ref, W2_ref, b2_ref,
                         go_ref, bo_ref)

  return pl.pallas_call(
      body,
      out_shape=jax.ShapeDtypeStruct((n, dout), jnp.float32),
  )(h, p, W1, b1.reshape(1, -1), g1.reshape(1, -1), be1.reshape(1, -1),
    W2, b2.reshape(1, -1), go.reshape(1, -1), bo.reshape(1, -1))


def _tc_layer0(h, plo, phi, W1, b1, g1, be1, W2, b2, go, bo):
  n = h.shape[0]
  dout = W2.shape[1]

  def body(h_ref, plo_ref, phi_ref, W1_ref, b1_ref, g1_ref, be1_ref, W2_ref,
           b2_ref, go_ref, bo_ref, o_ref):
    agg = jnp.concatenate(
        [_unpack_agg(plo_ref), _unpack_agg(phi_ref)], axis=1)
    m = h_ref[...] + agg
    o_ref[...] = _mlp_bn(m, W1_ref, b1_ref, g1_ref, be1_ref, W2_ref, b2_ref,
                         go_ref, bo_ref)

  return pl.pallas_call(
      body,
      out_shape=jax.ShapeDtypeStruct((n, dout), jnp.float32),
  )(h, plo, phi, W1, b1.reshape(1, -1), g1.reshape(1, -1),
    be1.reshape(1, -1), W2, b2.reshape(1, -1), go.reshape(1, -1),
    bo.reshape(1, -1))


def _tc_layer_readout(h, p, batch_row, W1, b1, g1, be1, W2, b2, go, bo):
  n = h.shape[0]
  dout = W2.shape[1]

  def body(h_ref, p_ref, batch_ref, W1_ref, b1_ref, g1_ref, be1_ref, W2_ref,
           b2_ref, go_ref, bo_ref, o_ref):
    m = h_ref[...] + _unpack_agg(p_ref)
    hout = _mlp_bn(m, W1_ref, b1_ref, g1_ref, be1_ref, W2_ref, b2_ref,
                   go_ref, bo_ref)
    gids = lax.broadcasted_iota(jnp.int32, (N_GRAPHS, n), 0)
    onehot = jnp.where(gids == batch_ref[...], 1.0, 0.0).astype(jnp.float32)
    sums = jnp.dot(onehot, hout, preferred_element_type=jnp.float32)
    counts = jnp.sum(onehot, axis=1, keepdims=True)
    o_ref[...] = sums / jnp.maximum(counts, 1.0)

  return pl.pallas_call(
      body,
      out_shape=jax.ShapeDtypeStruct((N_GRAPHS, dout), jnp.float32),
  )(h, p, batch_row, W1, b1.reshape(1, -1), g1.reshape(1, -1),
    be1.reshape(1, -1), W2, b2.reshape(1, -1), go.reshape(1, -1),
    bo.reshape(1, -1))


_agg_lo = _make_sc_agg(0)
_agg_hi = _make_sc_agg(64)


def kernel(x, edge_index, batch,
           W1_0, b1_0, g1_0, be1_0, W2_0, b2_0, go_0, bo_0,
           W1_1, b1_1, g1_1, be1_1, W2_1, b2_1, go_1, bo_1,
           W1_2, b1_2, g1_2, be1_2, W2_2, b2_2, go_2, bo_2):
  eii = edge_index.astype(jnp.int32)
  half = N_PAD // 2
  dst_r = (eii[1] % half) * 2 + eii[1] // half
  ei = jnp.stack([eii[0], dst_r]).reshape(2, NCHTOT, K)
  batch_row = batch.astype(jnp.int32).reshape(1, N_NODES)
  z64 = jnp.zeros((RPT, 64), jnp.float32)

  pk = (NC, N_PAD // 2, 128)
  p0_lo = _agg_lo(x, ei, z64).reshape(pk)
  p0_hi = _agg_hi(x, ei, z64).reshape(pk)
  h1 = _tc_layer0(x, p0_lo, p0_hi, W1_0, b1_0, g1_0, be1_0, W2_0, b2_0,
                  go_0, bo_0)
  p1 = _agg_lo(h1, ei, z64).reshape(pk)
  h2 = _tc_layer(h1, p1, W1_1, b1_1, g1_1, be1_1, W2_1, b2_1, go_1, bo_1)
  p2 = _agg_lo(h2, ei, z64).reshape(pk)
  return _tc_layer_readout(h2, p2, batch_row, W1_2, b1_2, g1_2, be1_2,
                           W2_2, b2_2, go_2, bo_2)
